# baseline (speedup 1.0000x reference)
import functools

import jax
import jax.numpy as jnp
from jax import lax
from jax.experimental import pallas as pl
from jax.experimental.pallas import tpu as pltpu
from jax.experimental.pallas import tpu_sc as plsc

_LANES = 16


def _sc_geometry():
    try:
        info = plsc.get_sparse_core_info()
        return info.num_cores, info.num_subcores
    except Exception:
        return 2, 16


def kernel(inputs, token_table, pos_table):
    batch, seq = inputs.shape
    vocab, emb = token_table.shape

    nc, ns = _sc_geometry()
    nw = nc * ns
    bc = batch // nw
    n_rt = emb // 8

    idx_t = inputs.T.astype(jnp.int32)

    mesh = plsc.VectorSubcoreMesh(core_axis_name="c", subcore_axis_name="s")

    @functools.partial(
        pl.kernel,
        out_type=jax.ShapeDtypeStruct((seq, n_rt, nw, 8 * 128), jnp.float32),
        mesh=mesh,
        scratch_types=[
            pltpu.VMEM((seq, bc), jnp.int32),
            pltpu.VMEM((2, bc, emb), jnp.float32),
            pltpu.VMEM((emb * 144,), jnp.float32),
            pltpu.VMEM((2, n_rt, 8 * 128), jnp.float32),
            pltpu.VMEM((seq, emb), jnp.float32),
            pltpu.SemaphoreType.DMA,
            pltpu.SemaphoreType.DMA,
            pltpu.SemaphoreType.DMA,
        ],
        compiler_params=pltpu.CompilerParams(use_tc_tiling_on_sc=False,
                                             needs_layout_passes=False),
    )
    def sc_kernel(idx_hbm, tok_hbm, pos_hbm, out_hbm, idx_v, rows2_v, tsk_v,
                  tout2_v, pos_v, gsem, osem0, osem1):
        wid = lax.axis_index("s") * nc + lax.axis_index("c")
        b0 = wid * bc
        pltpu.sync_copy(pos_hbm, pos_v)
        pltpu.sync_copy(idx_hbm.at[:, pl.ds(b0, bc)], idx_v)
        osems = (osem0, osem1)

        lane = lax.iota(jnp.int32, _LANES)
        base_sk = [lane * 145 + g * _LANES * 144
                   for g in range(emb // _LANES)]

        pltpu.async_copy(tok_hbm.at[idx_v.at[0]], rows2_v.at[0], gsem)

        def s2_body(s2, carry):
            for p in range(2):
                s = s2 * 2 + p
                rows_v = rows2_v.at[p]
                tout_v = tout2_v.at[p]
                pltpu.make_async_copy(tok_hbm.at[pl.ds(0, bc)], rows_v,
                                      gsem).wait()
                @pl.when(s < seq - 1)
                def _():
                    pltpu.async_copy(tok_hbm.at[idx_v.at[s + 1]],
                                     rows2_v.at[1 - p], gsem)
                @pl.when(s2 > 0)
                def _():
                    pltpu.make_async_copy(out_hbm.at[0, :, 0], tout_v,
                                          osems[p]).wait()

                pos_g = [pos_v[s, pl.ds(g * _LANES, _LANES)]
                         for g in range(emb // _LANES)]

                @plsc.parallel_loop(0, bc, step=8)
                def _(l0):
                    for u in range(8):
                        l = l0 + u
                        for g in range(emb // _LANES):
                            vals = (rows_v[l, pl.ds(g * _LANES, _LANES)]
                                    + pos_g[g])
                            plsc.store_scatter(tsk_v, [base_sk[g] + l], vals)
                @plsc.parallel_loop(0, emb, step=2)
                def _(e0):
                    for u2 in range(2):
                        e = e0 + u2
                        src = e * 144 + lax.bitwise_and(e, _LANES - 1)
                        rt = lax.shift_right_logical(e, 3)
                        dst = lax.shift_left(lax.bitwise_and(e, 7), 7)
                        for g2 in range(128 // _LANES):
                            tout_v[rt, pl.ds(dst + g2 * _LANES, _LANES)] = (
                                tsk_v[pl.ds(src + g2 * _LANES, _LANES)])
                pltpu.async_copy(tout_v, out_hbm.at[s, :, wid], osems[p])
            return carry

        lax.fori_loop(0, seq // 2, s2_body, 0)
        pltpu.make_async_copy(out_hbm.at[0, :, 0], tout2_v.at[0], osem0).wait()
        pltpu.make_async_copy(out_hbm.at[0, :, 0], tout2_v.at[1], osem1).wait()

    out = sc_kernel(idx_t, token_table, pos_table)
    out = out.reshape(seq, n_rt, nw, 8, 128)
    return jnp.transpose(out, (2, 4, 0, 1, 3)).reshape(batch, seq, emb)


# speedup vs baseline: 1.0110x; 1.0110x over previous
import functools

import jax
import jax.numpy as jnp
from jax import lax
from jax.experimental import pallas as pl
from jax.experimental.pallas import tpu as pltpu
from jax.experimental.pallas import tpu_sc as plsc

_LANES = 16


def _sc_geometry():
    try:
        info = plsc.get_sparse_core_info()
        return info.num_cores, info.num_subcores
    except Exception:
        return 2, 16


def kernel(inputs, token_table, pos_table):
    batch, seq = inputs.shape
    vocab, emb = token_table.shape

    nc, ns = _sc_geometry()
    nw = nc * ns
    bc = batch // nw
    n_rt = emb // 8

    idx_t = inputs.T.astype(jnp.int32)

    mesh = plsc.VectorSubcoreMesh(core_axis_name="c", subcore_axis_name="s")

    @functools.partial(
        pl.kernel,
        out_type=jax.ShapeDtypeStruct((seq, n_rt, nw, 8 * 128), jnp.float32),
        mesh=mesh,
        scratch_types=[
            pltpu.VMEM((seq, bc), jnp.int32),
            pltpu.VMEM((2, bc, emb), jnp.float32),
            pltpu.VMEM((emb * 144,), jnp.float32),
            pltpu.VMEM((2, n_rt, 8 * 128), jnp.float32),
            pltpu.VMEM((seq, emb), jnp.float32),
            pltpu.SemaphoreType.DMA,
            pltpu.SemaphoreType.DMA,
            pltpu.SemaphoreType.DMA,
        ],
        compiler_params=pltpu.CompilerParams(use_tc_tiling_on_sc=False,
                                             needs_layout_passes=False),
    )
    def sc_kernel(idx_hbm, tok_hbm, pos_hbm, out_hbm, idx_v, rows2_v, tsk_v,
                  tout2_v, pos_v, gsem, osem0, osem1):
        wid = lax.axis_index("s") * nc + lax.axis_index("c")
        b0 = wid * bc
        pltpu.sync_copy(pos_hbm, pos_v)
        pltpu.sync_copy(idx_hbm.at[:, pl.ds(b0, bc)], idx_v)
        osems = (osem0, osem1)

        lane = lax.iota(jnp.int32, _LANES)
        base_sk = [lane * 145 + g * _LANES * 144
                   for g in range(emb // _LANES)]

        pltpu.async_copy(tok_hbm.at[idx_v.at[0]], rows2_v.at[0], gsem)

        def s2_body(s2, carry):
            for p in range(2):
                s = s2 * 2 + p
                rows_v = rows2_v.at[p]
                tout_v = tout2_v.at[p]
                pltpu.make_async_copy(tok_hbm.at[pl.ds(0, bc)], rows_v,
                                      gsem).wait()
                @pl.when(s < seq - 1)
                def _():
                    pltpu.async_copy(tok_hbm.at[idx_v.at[s + 1]],
                                     rows2_v.at[1 - p], gsem)
                @pl.when(s2 > 0)
                def _():
                    pltpu.make_async_copy(out_hbm.at[0, :, 0], tout_v,
                                          osems[p]).wait()

                pos_g = [pos_v[s, pl.ds(g * _LANES, _LANES)]
                         for g in range(emb // _LANES)]

                @plsc.parallel_loop(0, bc, step=4)
                def _(l0):
                    for u in range(4):
                        l = l0 + u
                        for g in range(emb // _LANES):
                            vals = (rows_v[l, pl.ds(g * _LANES, _LANES)]
                                    + pos_g[g])
                            plsc.store_scatter(tsk_v, [base_sk[g] + l], vals)
                @plsc.parallel_loop(0, emb, step=2)
                def _(e0):
                    for u2 in range(2):
                        e = e0 + u2
                        src = e * 144 + lax.bitwise_and(e, _LANES - 1)
                        rt = lax.shift_right_logical(e, 3)
                        dst = lax.shift_left(lax.bitwise_and(e, 7), 7)
                        for g2 in range(128 // _LANES):
                            tout_v[rt, pl.ds(dst + g2 * _LANES, _LANES)] = (
                                tsk_v[pl.ds(src + g2 * _LANES, _LANES)])
                pltpu.async_copy(tout_v, out_hbm.at[s, :, wid], osems[p])
            return carry

        lax.fori_loop(0, seq // 2, s2_body, 0)
        pltpu.make_async_copy(out_hbm.at[0, :, 0], tout2_v.at[0], osem0).wait()
        pltpu.make_async_copy(out_hbm.at[0, :, 0], tout2_v.at[1], osem1).wait()

    out = sc_kernel(idx_t, token_table, pos_table)
    out = out.reshape(seq, n_rt, nw, 8, 128)
    return jnp.transpose(out, (2, 4, 0, 1, 3)).reshape(batch, seq, emb)


# speedup vs baseline: 1.0114x; 1.0004x over previous
import functools

import jax
import jax.numpy as jnp
from jax import lax
from jax.experimental import pallas as pl
from jax.experimental.pallas import tpu as pltpu
from jax.experimental.pallas import tpu_sc as plsc

_LANES = 16


def _sc_geometry():
    try:
        info = plsc.get_sparse_core_info()
        return info.num_cores, info.num_subcores
    except Exception:
        return 2, 16


def kernel(inputs, token_table, pos_table):
    batch, seq = inputs.shape
    vocab, emb = token_table.shape

    nc, ns = _sc_geometry()
    nw = nc * ns
    bc = batch // nw
    n_rt = emb // 8

    n_st = seq // 8
    idx_t = (inputs.astype(jnp.int32).T.reshape(n_st, 8, nw, bc)
             .transpose(0, 2, 1, 3))

    mesh = plsc.VectorSubcoreMesh(core_axis_name="c", subcore_axis_name="s")

    @functools.partial(
        pl.kernel,
        out_type=jax.ShapeDtypeStruct((seq, n_rt, nw, 8 * 128), jnp.float32),
        mesh=mesh,
        scratch_types=[
            pltpu.VMEM((n_st, 8, bc), jnp.int32),
            pltpu.VMEM((2, bc, emb), jnp.float32),
            pltpu.VMEM((emb * 144,), jnp.float32),
            pltpu.VMEM((2, n_rt, 8 * 128), jnp.float32),
            pltpu.VMEM((seq, emb), jnp.float32),
            pltpu.SemaphoreType.DMA,
            pltpu.SemaphoreType.DMA,
            pltpu.SemaphoreType.DMA,
        ],
        compiler_params=pltpu.CompilerParams(use_tc_tiling_on_sc=False,
                                             needs_layout_passes=False),
    )
    def sc_kernel(idx_hbm, tok_hbm, pos_hbm, out_hbm, idx_v, rows2_v, tsk_v,
                  tout2_v, pos_v, gsem, osem0, osem1):
        wid = lax.axis_index("s") * nc + lax.axis_index("c")
        b0 = wid * bc
        pltpu.sync_copy(pos_hbm, pos_v)
        pltpu.sync_copy(idx_hbm.at[:, wid], idx_v)
        osems = (osem0, osem1)

        lane = lax.iota(jnp.int32, _LANES)
        base_sk = [lane * 145 + g * _LANES * 144
                   for g in range(emb // _LANES)]

        pltpu.async_copy(tok_hbm.at[idx_v.at[0, 0]], rows2_v.at[0], gsem)

        def s2_body(s2, carry):
            for p in range(2):
                s = s2 * 2 + p
                rows_v = rows2_v.at[p]
                tout_v = tout2_v.at[p]
                pltpu.make_async_copy(tok_hbm.at[pl.ds(0, bc)], rows_v,
                                      gsem).wait()
                @pl.when(s < seq - 1)
                def _():
                    s1 = s + 1
                    pltpu.async_copy(
                        tok_hbm.at[idx_v.at[lax.shift_right_logical(s1, 3),
                                            lax.bitwise_and(s1, 7)]],
                        rows2_v.at[1 - p], gsem)
                @pl.when(s2 > 0)
                def _():
                    pltpu.make_async_copy(out_hbm.at[0, :, 0], tout_v,
                                          osems[p]).wait()

                pos_g = [pos_v[s, pl.ds(g * _LANES, _LANES)]
                         for g in range(emb // _LANES)]

                @plsc.parallel_loop(0, bc, step=4)
                def _(l0):
                    for u in range(4):
                        l = l0 + u
                        for g in range(emb // _LANES):
                            vals = (rows_v[l, pl.ds(g * _LANES, _LANES)]
                                    + pos_g[g])
                            plsc.store_scatter(tsk_v, [base_sk[g] + l], vals)
                @plsc.parallel_loop(0, emb, step=2)
                def _(e0):
                    for u2 in range(2):
                        e = e0 + u2
                        src = e * 144 + lax.bitwise_and(e, _LANES - 1)
                        rt = lax.shift_right_logical(e, 3)
                        dst = lax.shift_left(lax.bitwise_and(e, 7), 7)
                        for g2 in range(128 // _LANES):
                            tout_v[rt, pl.ds(dst + g2 * _LANES, _LANES)] = (
                                tsk_v[pl.ds(src + g2 * _LANES, _LANES)])
                pltpu.async_copy(tout_v, out_hbm.at[s, :, wid], osems[p])
            return carry

        lax.fori_loop(0, seq // 2, s2_body, 0)
        pltpu.make_async_copy(out_hbm.at[0, :, 0], tout2_v.at[0], osem0).wait()
        pltpu.make_async_copy(out_hbm.at[0, :, 0], tout2_v.at[1], osem1).wait()

    out = sc_kernel(idx_t, token_table, pos_table)
    out = out.reshape(seq, n_rt, nw, 8, 128)
    return jnp.transpose(out, (2, 4, 0, 1, 3)).reshape(batch, seq, emb)
